# Initial kernel scaffold; baseline (speedup 1.0000x reference)
import jax
import jax.numpy as jnp
from jax.experimental import pallas as pl


def kernel(hidden_states, Wq, bq, Wk, bk, Wv, bv, Wo, bo, mem_keys, mem_values, Wg, bg):
    raise NotImplementedError("write your pallas kernel here")


# speedup vs baseline: 3.3844x; 3.3844x over previous
import functools
import math

import jax
import jax.numpy as jnp
from jax.experimental import pallas as pl
from jax.experimental.pallas import tpu as pltpu

_S, _H = 2048, 1024
_NH, _NL, _MS, _TK = 16, 64, 512, 32
_HD = _H // _NH
_BLK = 256
_NEG = float("-inf")


def _landmark_kv_body(hs_ref, wk_ref, bk_ref, wv_ref, bv_ref,
                      k_ref, v_ref, p_ref):
    hs = hs_ref[...]
    imp = jnp.sqrt(jnp.sum(hs * hs, axis=1)).reshape(1, _S)
    iota = jax.lax.broadcasted_iota(jnp.int32, (1, _S), 1)
    work = imp
    for i in range(_NL):
        m = jnp.max(work, axis=1, keepdims=True)
        idx = jnp.min(jnp.where(work >= m, iota, _S), axis=1, keepdims=True)
        hit = iota == idx
        p_ref[i:i + 1, :] = hit.astype(jnp.float32)
        work = jnp.where(hit, _NEG, work)
    lm = jnp.dot(p_ref[...], hs, preferred_element_type=jnp.float32)
    k_ref[...] = jnp.dot(lm, wk_ref[...].T,
                         preferred_element_type=jnp.float32) + bk_ref[...]
    v_ref[...] = jnp.dot(lm, wv_ref[...].T,
                         preferred_element_type=jnp.float32) + bv_ref[...]


def _fused_body(hs_ref, wq_ref, bq_ref, wo_ref, bo_ref, mk_ref, mv_ref,
                wg_ref, bg_ref, k_ref, v_ref, out_ref):
    hs = hs_ref[...]
    q = jnp.dot(hs, wq_ref[...].T,
                preferred_element_type=jnp.float32) + bq_ref[...]
    k = k_ref[...]
    v = v_ref[...]
    scale = 1.0 / math.sqrt(_HD)
    ctx_parts = []
    for h in range(_NH):
        sl = slice(h * _HD, (h + 1) * _HD)
        qh = q[:, sl]
        kh = k[:, sl]
        vh = v[:, sl]
        s = jnp.dot(qh, kh.T, preferred_element_type=jnp.float32) * scale
        s = s - jnp.max(s, axis=1, keepdims=True)
        e = jnp.exp(s)
        a = e / jnp.sum(e, axis=1, keepdims=True)
        ctx_parts.append(jnp.dot(a, vh, preferred_element_type=jnp.float32))
    ctx = jnp.concatenate(ctx_parts, axis=1)
    att = jnp.dot(ctx, wo_ref[...].T,
                  preferred_element_type=jnp.float32) + bo_ref[...]
    ms = jnp.dot(att, mk_ref[...].T,
                 preferred_element_type=jnp.float32) * (1.0 / math.sqrt(_H))
    iota = jax.lax.broadcasted_iota(jnp.int32, (_BLK, _MS), 1)
    work = ms
    sel = jnp.zeros((_BLK, _MS), dtype=jnp.bool_)
    for _ in range(_TK):
        m = jnp.max(work, axis=1, keepdims=True)
        idx = jnp.min(jnp.where(work >= m, iota, _MS), axis=1, keepdims=True)
        hit = iota == idx
        sel = jnp.logical_or(sel, hit)
        work = jnp.where(hit, _NEG, work)
    masked = jnp.where(sel, ms, _NEG)
    mx = jnp.max(masked, axis=1, keepdims=True)
    e = jnp.exp(masked - mx)
    w = e / jnp.sum(e, axis=1, keepdims=True)
    mo = jnp.dot(w, mv_ref[...], preferred_element_type=jnp.float32)
    gate = jax.nn.sigmoid(
        jnp.sum(att * wg_ref[...], axis=1, keepdims=True) + bg_ref[0, 0])
    out_ref[...] = hs + att + gate * mo


def kernel(hidden_states, Wq, bq, Wk, bk, Wv, bv, Wo, bo,
           mem_keys, mem_values, Wg, bg):
    hs = hidden_states.reshape(_S, _H)
    bk2 = bk.reshape(1, _H)
    bv2 = bv.reshape(1, _H)
    bq2 = bq.reshape(1, _H)
    bo2 = bo.reshape(1, _H)
    wg2 = Wg.reshape(1, _H)
    bg2 = bg.reshape(1, 1)

    k, v = pl.pallas_call(
        _landmark_kv_body,
        out_shape=(
            jax.ShapeDtypeStruct((_NL, _H), jnp.float32),
            jax.ShapeDtypeStruct((_NL, _H), jnp.float32),
        ),
        scratch_shapes=[pltpu.VMEM((_NL, _S), jnp.float32)],
    )(hs, Wk, bk2, Wv, bv2)

    nblk = _S // _BLK
    full = lambda shape: pl.BlockSpec(shape, lambda i: (0, 0))
    out = pl.pallas_call(
        _fused_body,
        grid=(nblk,),
        in_specs=[
            pl.BlockSpec((_BLK, _H), lambda i: (i, 0)),
            full((_H, _H)),
            full((1, _H)),
            full((_H, _H)),
            full((1, _H)),
            full((_MS, _H)),
            full((_MS, _H)),
            full((1, _H)),
            full((1, 1)),
            full((_NL, _H)),
            full((_NL, _H)),
        ],
        out_specs=pl.BlockSpec((_BLK, _H), lambda i: (i, 0)),
        out_shape=jax.ShapeDtypeStruct((_S, _H), jnp.float32),
    )(hs, Wq, bq2, Wo, bo2, mem_keys, mem_values, wg2, bg2, k, v)
    return out.reshape(1, _S, _H)


# speedup vs baseline: 4.7716x; 1.4099x over previous
import functools
import math

import jax
import jax.numpy as jnp
from jax.experimental import pallas as pl
from jax.experimental.pallas import tpu as pltpu

_S, _H = 2048, 1024
_NH, _NL, _MS, _TK = 16, 64, 512, 32
_HD = _H // _NH
_BLK = 256
_NEG = float("-inf")


def _landmark_kv_body(hs3_ref, hs_ref, wk_ref, bk_ref, wv_ref, bv_ref,
                      k_ref, v_ref, lm_ref):
    hs3 = hs3_ref[...]
    imp = jnp.sqrt(jnp.sum(hs3 * hs3, axis=2))
    r_io = jax.lax.broadcasted_iota(jnp.int32, (16, 128), 0)
    c_io = jax.lax.broadcasted_iota(jnp.int32, (16, 128), 1)
    flat = r_io * 128 + c_io
    work = imp
    for i in range(_NL):
        m = jnp.max(work)
        hitmask = work >= m
        idx = jnp.min(jnp.where(hitmask, flat, _S))
        lm_ref[i:i + 1, :] = hs_ref[pl.ds(idx, 1), :]
        work = jnp.where(flat == idx, _NEG, work)
    lm = lm_ref[...].astype(jnp.bfloat16)
    k_ref[...] = jnp.dot(lm, wk_ref[...].T,
                         preferred_element_type=jnp.float32) + bk_ref[...]
    v_ref[...] = jnp.dot(lm, wv_ref[...].T,
                         preferred_element_type=jnp.float32) + bv_ref[...]


def _fused_body(hs_ref, wq_ref, bq_ref, wo_ref, bo_ref, mk_ref, mv_ref,
                wg_ref, bg_ref, k_ref, v_ref, out_ref):
    hs = hs_ref[...]
    hsb = hs.astype(jnp.bfloat16)
    q = jnp.dot(hsb, wq_ref[...].T,
                preferred_element_type=jnp.float32) + bq_ref[...]
    k = k_ref[...].astype(jnp.bfloat16)
    v = v_ref[...].astype(jnp.bfloat16)
    scale = 1.0 / math.sqrt(_HD)
    ctx_parts = []
    for h in range(_NH):
        sl = slice(h * _HD, (h + 1) * _HD)
        qh = q[:, sl].astype(jnp.bfloat16)
        kh = k[:, sl]
        vh = v[:, sl]
        s = jnp.dot(qh, kh.T, preferred_element_type=jnp.float32) * scale
        s = s - jnp.max(s, axis=1, keepdims=True)
        e = jnp.exp(s)
        a = (e / jnp.sum(e, axis=1, keepdims=True)).astype(jnp.bfloat16)
        ctx_parts.append(jnp.dot(a, vh, preferred_element_type=jnp.float32))
    ctx = jnp.concatenate(ctx_parts, axis=1).astype(jnp.bfloat16)
    att = jnp.dot(ctx, wo_ref[...].T,
                  preferred_element_type=jnp.float32) + bo_ref[...]
    attb = att.astype(jnp.bfloat16)
    ms = jnp.dot(attb, mk_ref[...].T,
                 preferred_element_type=jnp.float32) * (1.0 / math.sqrt(_H))
    work = ms
    for _ in range(_TK):
        m = jnp.max(work, axis=1, keepdims=True)
        work = jnp.where(work >= m, _NEG, work)
    sel = work == _NEG
    masked = jnp.where(sel, ms, _NEG)
    mx = jnp.max(masked, axis=1, keepdims=True)
    e = jnp.exp(masked - mx)
    w = (e / jnp.sum(e, axis=1, keepdims=True)).astype(jnp.bfloat16)
    mo = jnp.dot(w, mv_ref[...], preferred_element_type=jnp.float32)
    gate = jax.nn.sigmoid(
        jnp.sum(att * wg_ref[...], axis=1, keepdims=True) + bg_ref[0, 0])
    out_ref[...] = hs + att + gate * mo


def kernel(hidden_states, Wq, bq, Wk, bk, Wv, bv, Wo, bo,
           mem_keys, mem_values, Wg, bg):
    f32, bf16 = jnp.float32, jnp.bfloat16
    hs = hidden_states.reshape(_S, _H)
    hs3 = hidden_states.reshape(16, 128, _H)
    bk2 = bk.reshape(1, _H)
    bv2 = bv.reshape(1, _H)
    bq2 = bq.reshape(1, _H)
    bo2 = bo.reshape(1, _H)
    wg2 = Wg.reshape(1, _H)
    bg2 = bg.reshape(1, 1)

    k, v = pl.pallas_call(
        _landmark_kv_body,
        out_shape=(
            jax.ShapeDtypeStruct((_NL, _H), f32),
            jax.ShapeDtypeStruct((_NL, _H), f32),
        ),
        scratch_shapes=[pltpu.VMEM((_NL, _H), f32)],
    )(hs3, hs, Wk.astype(bf16), bk2, Wv.astype(bf16), bv2)

    nblk = _S // _BLK
    full = lambda shape: pl.BlockSpec(shape, lambda i: (0, 0))
    out = pl.pallas_call(
        _fused_body,
        grid=(nblk,),
        in_specs=[
            pl.BlockSpec((_BLK, _H), lambda i: (i, 0)),
            full((_H, _H)),
            full((1, _H)),
            full((_H, _H)),
            full((1, _H)),
            full((_MS, _H)),
            full((_MS, _H)),
            full((1, _H)),
            full((1, 1)),
            full((_NL, _H)),
            full((_NL, _H)),
        ],
        out_specs=pl.BlockSpec((_BLK, _H), lambda i: (i, 0)),
        out_shape=jax.ShapeDtypeStruct((_S, _H), f32),
    )(hs, Wq.astype(bf16), bq2, Wo.astype(bf16), bo2, mem_keys.astype(bf16),
      mem_values.astype(bf16), wg2, bg2, k, v)
    return out.reshape(1, _S, _H)


# speedup vs baseline: 4.8738x; 1.0214x over previous
import functools
import math

import jax
import jax.numpy as jnp
from jax.experimental import pallas as pl
from jax.experimental.pallas import tpu as pltpu

_S, _H = 2048, 1024
_NH, _NL, _MS, _TK = 16, 64, 512, 32
_HD = _H // _NH
_BLK = 256
_NEG = float("-inf")


def _landmark_kv_body(hs3_ref, wk_ref, bk_ref, wv_ref, bv_ref,
                      k_ref, v_ref, p_ref):
    hs3 = hs3_ref[...]
    imp = jnp.sqrt(jnp.sum(hs3 * hs3, axis=2))
    r_io = jax.lax.broadcasted_iota(jnp.int32, (16, 128), 0)
    c_io = jax.lax.broadcasted_iota(jnp.int32, (16, 128), 1)
    flat = r_io * 128 + c_io
    work = imp
    for i in range(_NL):
        m = jnp.max(work, axis=(0, 1), keepdims=True)
        idx = jnp.min(jnp.where(work >= m, flat, _S),
                      axis=(0, 1), keepdims=True)
        hit = flat == idx
        p_ref[i, :, :] = hit.astype(jnp.bfloat16)
        work = jnp.where(hit, _NEG, work)
    hs3b = hs3.astype(jnp.bfloat16)
    lm = jnp.zeros((_NL, _H), dtype=jnp.float32)
    for r in range(16):
        lm = lm + jnp.dot(p_ref[:, r, :], hs3b[r],
                          preferred_element_type=jnp.float32)
    lmb = lm.astype(jnp.bfloat16)
    k_ref[...] = jnp.dot(lmb, wk_ref[...].T,
                         preferred_element_type=jnp.float32) + bk_ref[...]
    v_ref[...] = jnp.dot(lmb, wv_ref[...].T,
                         preferred_element_type=jnp.float32) + bv_ref[...]


def _fused_body(hs_ref, wq_ref, bq_ref, wo_ref, bo_ref, mk_ref, mv_ref,
                wg_ref, bg_ref, k_ref, v_ref, out_ref):
    hs = hs_ref[...]
    hsb = hs.astype(jnp.bfloat16)
    q = jnp.dot(hsb, wq_ref[...].T,
                preferred_element_type=jnp.float32) + bq_ref[...]
    k = k_ref[...].astype(jnp.bfloat16)
    v = v_ref[...].astype(jnp.bfloat16)
    scale = 1.0 / math.sqrt(_HD)
    ctx_parts = []
    for h in range(_NH):
        sl = slice(h * _HD, (h + 1) * _HD)
        qh = q[:, sl].astype(jnp.bfloat16)
        kh = k[:, sl]
        vh = v[:, sl]
        s = jnp.dot(qh, kh.T, preferred_element_type=jnp.float32) * scale
        s = s - jnp.max(s, axis=1, keepdims=True)
        e = jnp.exp(s)
        a = (e / jnp.sum(e, axis=1, keepdims=True)).astype(jnp.bfloat16)
        ctx_parts.append(jnp.dot(a, vh, preferred_element_type=jnp.float32))
    ctx = jnp.concatenate(ctx_parts, axis=1).astype(jnp.bfloat16)
    att = jnp.dot(ctx, wo_ref[...].T,
                  preferred_element_type=jnp.float32) + bo_ref[...]
    attb = att.astype(jnp.bfloat16)
    ms = jnp.dot(attb, mk_ref[...].T,
                 preferred_element_type=jnp.float32) * (1.0 / math.sqrt(_H))
    work = ms
    for _ in range(_TK):
        m = jnp.max(work, axis=1, keepdims=True)
        work = jnp.where(work >= m, _NEG, work)
    sel = work == _NEG
    masked = jnp.where(sel, ms, _NEG)
    mx = jnp.max(masked, axis=1, keepdims=True)
    e = jnp.exp(masked - mx)
    w = (e / jnp.sum(e, axis=1, keepdims=True)).astype(jnp.bfloat16)
    mo = jnp.dot(w, mv_ref[...], preferred_element_type=jnp.float32)
    gate = jax.nn.sigmoid(
        jnp.sum(att * wg_ref[...], axis=1, keepdims=True) + bg_ref[0, 0])
    out_ref[...] = hs + att + gate * mo


def kernel(hidden_states, Wq, bq, Wk, bk, Wv, bv, Wo, bo,
           mem_keys, mem_values, Wg, bg):
    f32, bf16 = jnp.float32, jnp.bfloat16
    hs = hidden_states.reshape(_S, _H)
    hs3 = hidden_states.reshape(16, 128, _H)
    bk2 = bk.reshape(1, _H)
    bv2 = bv.reshape(1, _H)
    bq2 = bq.reshape(1, _H)
    bo2 = bo.reshape(1, _H)
    wg2 = Wg.reshape(1, _H)
    bg2 = bg.reshape(1, 1)

    k, v = pl.pallas_call(
        _landmark_kv_body,
        out_shape=(
            jax.ShapeDtypeStruct((_NL, _H), f32),
            jax.ShapeDtypeStruct((_NL, _H), f32),
        ),
        scratch_shapes=[pltpu.VMEM((_NL, 16, 128), bf16)],
    )(hs3, Wk.astype(bf16), bk2, Wv.astype(bf16), bv2)

    nblk = _S // _BLK
    full = lambda shape: pl.BlockSpec(shape, lambda i: (0, 0))
    out = pl.pallas_call(
        _fused_body,
        grid=(nblk,),
        in_specs=[
            pl.BlockSpec((_BLK, _H), lambda i: (i, 0)),
            full((_H, _H)),
            full((1, _H)),
            full((_H, _H)),
            full((1, _H)),
            full((_MS, _H)),
            full((_MS, _H)),
            full((1, _H)),
            full((1, 1)),
            full((_NL, _H)),
            full((_NL, _H)),
        ],
        out_specs=pl.BlockSpec((_BLK, _H), lambda i: (i, 0)),
        out_shape=jax.ShapeDtypeStruct((_S, _H), f32),
    )(hs, Wq.astype(bf16), bq2, Wo.astype(bf16), bo2, mem_keys.astype(bf16),
      mem_values.astype(bf16), wg2, bg2, k, v)
    return out.reshape(1, _S, _H)
